# baseline (speedup 1.0000x reference)
import numpy as np
import jax
import jax.numpy as jnp
from jax import lax
from jax.experimental import pallas as pl
from jax.experimental import pallas as _pl_unused
from jax.experimental.pallas import tpu as pltpu
from jax.experimental.pallas import tpu_sc as plsc

_FIELD_DIMS = [38461] * 26
_F = 26
_D = 16
_H = 2
_B = 16384
_NC, _NS = 2, 16
_NW = _NC * _NS
_NT = _B // 128
_TPW = _NT // _NW
_CB = 16
_NCHUNK = 128 // _CB
_CBF = _CB * _F
_NG = 4
_GR = _CBF // _NG
_DT = _F * _D // 8
_SLAB = _H * _DT * 1024

_off = np.concatenate([[0], np.cumsum(_FIELD_DIMS)[:-1]]).astype(np.int32)
_p = np.arange(_CBF)
_OFF_T = np.asarray(_off[_p % _F], dtype=np.int32)


def _body(w, xf, offt, o, xv, offv, idxv, bufr, slab, sem):
    wid = lax.axis_index("s") * _NC + lax.axis_index("c")
    pltpu.sync_copy(offt, offv)
    spat = lax.iota(jnp.int32, 16) * 128

    def tile_loop(tl, car):
        bt = wid * _TPW + tl

        def chunk(c, car2):
            b0 = bt * 128 + c * _CB
            pltpu.sync_copy(xf.at[pl.ds(b0 * _F, _CBF)], xv)

            def build(i, cc):
                s = pl.ds(pl.multiple_of(i * 16, 16), 16)
                idxv[s] = xv[s] + offv[s]
                return cc

            lax.fori_loop(0, _CBF // 16, build, 0)

            copies = [
                pltpu.async_copy(w.at[idxv.at[pl.ds(g * _GR, _GR)]],
                                 bufr.at[pl.ds(g * _GR, _GR)], sem)
                for g in range(_NG)
            ]
            for cp in copies:
                cp.wait()

            def row(bb, cc):
                bl = c * _CB + bb
                r = bb * _F
                for f in range(_F):
                    base = f * 2048 + bl
                    plsc.store_scatter(slab, [spat + base],
                                       bufr[r + f, pl.ds(0, _D)])
                    plsc.store_scatter(slab, [spat + (base + _DT * 1024)],
                                       bufr[r + f, pl.ds(_D, _D)])
                return cc

            lax.fori_loop(0, _CB, row, 0)
            return car2

        lax.fori_loop(0, _NCHUNK, chunk, 0)

        def outd(t, cc):
            pltpu.sync_copy(slab.at[pl.ds(t * 1024, 1024)], o.at[t, bt])
            return cc

        lax.fori_loop(0, _H * _DT, outd, 0)
        return car

    lax.fori_loop(0, _TPW, tile_loop, 0)


_launch = pl.kernel(
    _body,
    out_type=jax.ShapeDtypeStruct((_H * _DT, _NT, 1024), jnp.float32),
    mesh=plsc.VectorSubcoreMesh(core_axis_name="c", subcore_axis_name="s"),
    compiler_params=pltpu.CompilerParams(needs_layout_passes=False,
                                         use_tc_tiling_on_sc=False),
    scratch_types=[
        pltpu.VMEM((_CBF,), jnp.int32),
        pltpu.VMEM((_CBF,), jnp.int32),
        pltpu.VMEM((_CBF,), jnp.int32),
        pltpu.VMEM((_CBF, _H * _D), jnp.float32),
        pltpu.VMEM((_SLAB,), jnp.float32),
        pltpu.SemaphoreType.DMA,
    ],
)


@jax.jit
def kernel(x, W):
    tiles = _launch(W, x.reshape(-1), jnp.asarray(_OFF_T))
    t5 = tiles.reshape(_H, _DT, _NT, 8, 128)
    return t5.transpose(2, 4, 0, 1, 3).reshape(_B, _H, _F * _D)


# speedup vs baseline: 1.0552x; 1.0552x over previous
import numpy as np
import jax
import jax.numpy as jnp
from jax import lax
from jax.experimental import pallas as pl
from jax.experimental.pallas import tpu as pltpu
from jax.experimental.pallas import tpu_sc as plsc

_FIELD_DIMS = [38461] * 26
_F = 26
_D = 16
_H = 2
_B = 16384
_NC, _NS = 2, 16
_NW = _NC * _NS
_BPW = _B // _NW
_CB = 32
_NCHUNK = _BPW // _CB
_CBF = _CB * _F
_NG = 8
_GR = _CBF // _NG

_off = np.concatenate([[0], np.cumsum(_FIELD_DIMS)[:-1]]).astype(np.int32)
_p = np.arange(_CBF)
_OFF_T = np.asarray(_off[_p % _F], dtype=np.int32)


def _body(w, xf, offt, o, xva, xvb, offv, idxa, idxb, bufra, bufrb, bufo,
          sema, semb):
    wid = lax.axis_index("s") * _NC + lax.axis_index("c")
    pltpu.sync_copy(offt, offv)

    xv = [xva, xvb]
    idxv = [idxa, idxb]
    bufr = [bufra, bufrb]
    sem = [sema, semb]

    def fire(c):
        p = c % 2
        b0 = wid * _BPW + c * _CB
        pltpu.sync_copy(xf.at[pl.ds(b0 * _F, _CBF)], xv[p])

        def build(i, carry):
            s = pl.ds(pl.multiple_of(i * 16, 16), 16)
            idxv[p][s] = xv[p][s] + offv[s]
            return carry

        lax.fori_loop(0, _CBF // 16, build, 0)
        return [
            pltpu.async_copy(w.at[idxv[p].at[pl.ds(g * _GR, _GR)]],
                             bufr[p].at[pl.ds(g * _GR, _GR)], sem[p])
            for g in range(_NG)
        ]

    copies = fire(0)
    for c in range(_NCHUNK):
        nxt = fire(c + 1) if c + 1 < _NCHUNK else None
        for cp in copies:
            cp.wait()
        p = c % 2
        src = bufr[p]

        def split(bb, carry):
            r = bb * _F
            for f in range(_F):
                bufo[bb, 0, pl.ds(f * _D, _D)] = src[r + f, pl.ds(0, _D)]
                bufo[bb, 1, pl.ds(f * _D, _D)] = src[r + f, pl.ds(_D, _D)]
            return carry

        lax.fori_loop(0, _CB, split, 0)
        b0 = wid * _BPW + c * _CB
        pltpu.sync_copy(bufo, o.at[pl.ds(b0, _CB)])
        copies = nxt


_launch = pl.kernel(
    _body,
    out_type=jax.ShapeDtypeStruct((_B, _H, _F * _D), jnp.float32),
    mesh=plsc.VectorSubcoreMesh(core_axis_name="c", subcore_axis_name="s"),
    compiler_params=pltpu.CompilerParams(needs_layout_passes=False,
                                         use_tc_tiling_on_sc=False),
    scratch_types=[
        pltpu.VMEM((_CBF,), jnp.int32),
        pltpu.VMEM((_CBF,), jnp.int32),
        pltpu.VMEM((_CBF,), jnp.int32),
        pltpu.VMEM((_CBF,), jnp.int32),
        pltpu.VMEM((_CBF,), jnp.int32),
        pltpu.VMEM((_CBF, _H * _D), jnp.float32),
        pltpu.VMEM((_CBF, _H * _D), jnp.float32),
        pltpu.VMEM((_CB, _H, _F * _D), jnp.float32),
        pltpu.SemaphoreType.DMA,
        pltpu.SemaphoreType.DMA,
    ],
)


@jax.jit
def kernel(x, W):
    return _launch(W, x.reshape(-1), jnp.asarray(_OFF_T))


# speedup vs baseline: 1.0680x; 1.0122x over previous
import numpy as np
import jax
import jax.numpy as jnp
from jax import lax
from jax.experimental import pallas as pl
from jax.experimental.pallas import tpu as pltpu
from jax.experimental.pallas import tpu_sc as plsc

_FIELD_DIMS = [38461] * 26
_F = 26
_D = 16
_H = 2
_B = 16384
_NC, _NS = 2, 16
_NW = _NC * _NS
_BPW = _B // _NW
_CB = 32
_NCHUNK = _BPW // _CB
_CBF = _CB * _F
_NG = 8
_GR = _CBF // _NG

_off = np.concatenate([[0], np.cumsum(_FIELD_DIMS)[:-1]]).astype(np.int32)
_p = np.arange(_CBF)
_OFF_T = np.asarray(_off[_p % _F], dtype=np.int32)


def _body(w, xf, offt, o, xva, xvb, offv, idxa, idxb, bufra, bufrb,
          bufoa, bufob, sema, semb, osema, osemb):
    wid = lax.axis_index("s") * _NC + lax.axis_index("c")
    pltpu.sync_copy(offt, offv)

    xv = [xva, xvb]
    idxv = [idxa, idxb]
    bufr = [bufra, bufrb]
    bufo = [bufoa, bufob]
    sem = [sema, semb]
    osem = [osema, osemb]

    def fire(c):
        p = c % 2
        b0 = wid * _BPW + c * _CB
        pltpu.sync_copy(xf.at[pl.ds(b0 * _F, _CBF)], xv[p])

        def build(i, carry):
            s = pl.ds(pl.multiple_of(i * 16, 16), 16)
            idxv[p][s] = xv[p][s] + offv[s]
            return carry

        lax.fori_loop(0, _CBF // 16, build, 0)
        return [
            pltpu.async_copy(w.at[idxv[p].at[pl.ds(g * _GR, _GR)]],
                             bufr[p].at[pl.ds(g * _GR, _GR)], sem[p])
            for g in range(_NG)
        ]

    copies = fire(0)
    out_pending = [None, None]
    for c in range(_NCHUNK):
        nxt = fire(c + 1) if c + 1 < _NCHUNK else None
        for cp in copies:
            cp.wait()
        p = c % 2
        if out_pending[p] is not None:
            out_pending[p].wait()
        src = bufr[p]
        dst = bufo[p]

        def split(bb, carry):
            r = bb * _F
            for f in range(_F):
                dst[bb, 0, pl.ds(f * _D, _D)] = src[r + f, pl.ds(0, _D)]
                dst[bb, 1, pl.ds(f * _D, _D)] = src[r + f, pl.ds(_D, _D)]
            return carry

        lax.fori_loop(0, _CB, split, 0)
        b0 = wid * _BPW + c * _CB
        out_pending[p] = pltpu.async_copy(dst, o.at[pl.ds(b0, _CB)], osem[p])
        copies = nxt

    for q in range(2):
        if out_pending[q] is not None:
            out_pending[q].wait()


_launch = pl.kernel(
    _body,
    out_type=jax.ShapeDtypeStruct((_B, _H, _F * _D), jnp.float32),
    mesh=plsc.VectorSubcoreMesh(core_axis_name="c", subcore_axis_name="s"),
    compiler_params=pltpu.CompilerParams(needs_layout_passes=False,
                                         use_tc_tiling_on_sc=False),
    scratch_types=[
        pltpu.VMEM((_CBF,), jnp.int32),
        pltpu.VMEM((_CBF,), jnp.int32),
        pltpu.VMEM((_CBF,), jnp.int32),
        pltpu.VMEM((_CBF,), jnp.int32),
        pltpu.VMEM((_CBF,), jnp.int32),
        pltpu.VMEM((_CBF, _H * _D), jnp.float32),
        pltpu.VMEM((_CBF, _H * _D), jnp.float32),
        pltpu.VMEM((_CB, _H, _F * _D), jnp.float32),
        pltpu.VMEM((_CB, _H, _F * _D), jnp.float32),
        pltpu.SemaphoreType.DMA,
        pltpu.SemaphoreType.DMA,
        pltpu.SemaphoreType.DMA,
        pltpu.SemaphoreType.DMA,
    ],
)


@jax.jit
def kernel(x, W):
    return _launch(W, x.reshape(-1), jnp.asarray(_OFF_T))
